# Initial kernel scaffold; baseline (speedup 1.0000x reference)
import jax
import jax.numpy as jnp
from jax.experimental import pallas as pl


def kernel(x, edge_index, W1, b1, W2, b2):
    raise NotImplementedError("write your pallas kernel here")


# speedup vs baseline: 13.1199x; 13.1199x over previous
import functools

import jax
import jax.numpy as jnp
from jax import lax
from jax.experimental import pallas as pl
from jax.experimental.pallas import tpu as pltpu
from jax.experimental.pallas import tpu_sc as plsc

N_NODES = 10000
D = 128
N_EDGES = 320000

NC = 2
NS = 16
NW = NC * NS
EPW = N_EDGES // NW
CH = 80
NCHUNK = EPW // CH
RPT = N_NODES // NS

_mesh = plsc.VectorSubcoreMesh(core_axis_name="c", subcore_axis_name="s")


@functools.partial(
    pl.kernel,
    mesh=_mesh,
    compiler_params=pltpu.CompilerParams(use_tc_tiling_on_sc=False),
    out_type=jax.ShapeDtypeStruct((NC, N_NODES, 16), jnp.float32),
    scratch_types=[
        pltpu.VMEM((CH,), jnp.int32),
        pltpu.VMEM((CH, 16), jnp.float32),
        pltpu.VMEM((RPT, 16), jnp.float32),
        pltpu.VMEM_SHARED((N_NODES, 16), jnp.float32),
    ],
)
def _sc_deg(dst_hbm, out_hbm, idx_v, ones_v, z_v, deg_sh):
    cid = lax.axis_index("c")
    sid = lax.axis_index("s")
    wid = sid * NC + cid

    def fill_ones(i, _):
        ones_v[i] = jnp.full((16,), 1.0, jnp.float32)
        return _

    lax.fori_loop(0, CH, fill_ones, None)

    def fill_z(i, _):
        z_v[i] = jnp.zeros((16,), jnp.float32)
        return _

    lax.fori_loop(0, RPT, fill_z, None)
    pltpu.sync_copy(z_v, deg_sh.at[pl.ds(sid * RPT, RPT)])
    plsc.subcore_barrier()

    def chunk(c, _):
        pltpu.sync_copy(dst_hbm.at[wid, c], idx_v)
        pltpu.sync_copy(ones_v, deg_sh.at[idx_v], add=True)
        return _

    lax.fori_loop(0, NCHUNK, chunk, None)
    plsc.subcore_barrier()
    pltpu.sync_copy(deg_sh.at[pl.ds(sid * RPT, RPT)],
                    out_hbm.at[cid, pl.ds(sid * RPT, RPT)])


ZROWS = 125


@functools.partial(
    pl.kernel,
    mesh=_mesh,
    compiler_params=pltpu.CompilerParams(use_tc_tiling_on_sc=False),
    out_type=jax.ShapeDtypeStruct((NC, N_NODES, D), jnp.float32),
    scratch_types=[
        pltpu.VMEM((CH,), jnp.int32),
        pltpu.VMEM((CH,), jnp.int32),
        pltpu.VMEM((CH, D), jnp.float32),
        pltpu.VMEM((ZROWS, D), jnp.float32),
        pltpu.VMEM_SHARED((N_NODES, D), jnp.float32),
        pltpu.SemaphoreType.DMA,
    ],
)
def _sc_agg(p_hbm, src_hbm, dst_hbm, out_hbm,
            src_v, dst_v, rows_v, z_v, agg_sh, sem):
    cid = lax.axis_index("c")
    sid = lax.axis_index("s")
    wid = sid * NC + cid

    def fill(i, _):
        for j in range(D // 16):
            z_v[i, pl.ds(16 * j, 16)] = jnp.zeros((16,), jnp.float32)
        return _

    lax.fori_loop(0, ZROWS, fill, None)
    for r in range(RPT // ZROWS):
        pltpu.sync_copy(z_v, agg_sh.at[pl.ds(sid * RPT + r * ZROWS, ZROWS)])
    plsc.subcore_barrier()

    def chunk(c, _):
        pltpu.sync_copy(src_hbm.at[wid, c], src_v)
        pltpu.sync_copy(dst_hbm.at[wid, c], dst_v)
        pltpu.async_copy(p_hbm.at[src_v], rows_v, sem).wait()
        pltpu.sync_copy(rows_v, agg_sh.at[dst_v], add=True)
        return _

    lax.fori_loop(0, NCHUNK, chunk, None)
    plsc.subcore_barrier()
    pltpu.sync_copy(agg_sh.at[pl.ds(sid * RPT, RPT)],
                    out_hbm.at[cid, pl.ds(sid * RPT, RPT)])


_R = 1000


def _prep_body(x_ref, w_ref, degp_ref, p_ref, dis_ref):
    h = jnp.dot(x_ref[...], w_ref[...], preferred_element_type=jnp.float32)
    dis = lax.rsqrt(degp_ref[0] + degp_ref[1] + 1.0)
    dis_ref[...] = dis
    p_ref[...] = h * dis[:, :1]


def _mid_body(parts_ref, p1_ref, dis_ref, b1_ref, w2_ref, p2_ref):
    dis = dis_ref[...][:, :1]
    t = dis * (parts_ref[0] + parts_ref[1] + p1_ref[...]) + b1_ref[...]
    h2 = jnp.maximum(t, 0.0)
    p2_ref[...] = jnp.dot(h2, w2_ref[...],
                          preferred_element_type=jnp.float32) * dis


def _out_body(parts_ref, p2_ref, dis_ref, b2_ref, out_ref):
    dis = dis_ref[...][:, :1]
    out_ref[...] = dis * (parts_ref[0] + parts_ref[1] + p2_ref[...]) \
        + b2_ref[...]


def _tc_prep(x, W1, degp):
    grid = (N_NODES // _R,)
    return pl.pallas_call(
        _prep_body,
        grid=grid,
        in_specs=[
            pl.BlockSpec((_R, D), lambda i: (i, 0)),
            pl.BlockSpec((D, D), lambda i: (0, 0)),
            pl.BlockSpec((NC, _R, 16), lambda i: (0, i, 0)),
        ],
        out_specs=[
            pl.BlockSpec((_R, D), lambda i: (i, 0)),
            pl.BlockSpec((_R, 16), lambda i: (i, 0)),
        ],
        out_shape=[
            jax.ShapeDtypeStruct((N_NODES, D), jnp.float32),
            jax.ShapeDtypeStruct((N_NODES, 16), jnp.float32),
        ],
    )(x, W1, degp)


def _tc_mid(parts, p1, dis16, b1, W2):
    grid = (N_NODES // _R,)
    return pl.pallas_call(
        _mid_body,
        grid=grid,
        in_specs=[
            pl.BlockSpec((NC, _R, D), lambda i: (0, i, 0)),
            pl.BlockSpec((_R, D), lambda i: (i, 0)),
            pl.BlockSpec((_R, 16), lambda i: (i, 0)),
            pl.BlockSpec((1, D), lambda i: (0, 0)),
            pl.BlockSpec((D, D), lambda i: (0, 0)),
        ],
        out_specs=pl.BlockSpec((_R, D), lambda i: (i, 0)),
        out_shape=jax.ShapeDtypeStruct((N_NODES, D), jnp.float32),
    )(parts, p1, dis16, b1, W2)


def _tc_out(parts, p2, dis16, b2):
    grid = (N_NODES // _R,)
    return pl.pallas_call(
        _out_body,
        grid=grid,
        in_specs=[
            pl.BlockSpec((NC, _R, D), lambda i: (0, i, 0)),
            pl.BlockSpec((_R, D), lambda i: (i, 0)),
            pl.BlockSpec((_R, 16), lambda i: (i, 0)),
            pl.BlockSpec((1, D), lambda i: (0, 0)),
        ],
        out_specs=pl.BlockSpec((_R, D), lambda i: (i, 0)),
        out_shape=jax.ShapeDtypeStruct((N_NODES, D), jnp.float32),
    )(parts, p2, dis16, b2)


def kernel(x, edge_index, W1, b1, W2, b2):
    ei = edge_index.astype(jnp.int32)
    src3 = ei[0].reshape(NW, NCHUNK, CH)
    dst3 = ei[1].reshape(NW, NCHUNK, CH)

    degp = _sc_deg(dst3)
    p1, dis16 = _tc_prep(x, W1, degp)
    parts1 = _sc_agg(p1, src3, dst3)
    p2 = _tc_mid(parts1, p1, dis16, b1.reshape(1, D), W2)
    parts2 = _sc_agg(p2, src3, dst3)
    return _tc_out(parts2, p2, dis16, b2.reshape(1, D))


# speedup vs baseline: 29.3864x; 2.2398x over previous
import functools

import jax
import jax.numpy as jnp
from jax import lax
from jax.experimental import pallas as pl
from jax.experimental.pallas import tpu as pltpu
from jax.experimental.pallas import tpu_sc as plsc

N_NODES = 10000
D = 128
N_EDGES = 320000

NC = 2
NS = 16
NW = NC * NS
NPAD = 10240
CH = 80
NCHUNK = 128
EPW = NCHUNK * CH
EPAD = NW * EPW
RPT = NPAD // NS
NBUF = 2

_mesh = plsc.VectorSubcoreMesh(core_axis_name="c", subcore_axis_name="s")
_sc_params = pltpu.CompilerParams(use_tc_tiling_on_sc=False)


@functools.partial(
    pl.kernel,
    mesh=_mesh,
    compiler_params=_sc_params,
    out_type=jax.ShapeDtypeStruct((NC, NPAD, 16), jnp.float32),
    scratch_types=[
        pltpu.VMEM((NCHUNK, CH), jnp.int32),
        pltpu.VMEM((CH, 16), jnp.float32),
        pltpu.VMEM((RPT, 16), jnp.float32),
        pltpu.VMEM_SHARED((NPAD, 16), jnp.float32),
    ],
)
def _sc_deg(dst_hbm, out_hbm, dst_all, ones_v, z_v, deg_sh):
    cid = lax.axis_index("c")
    sid = lax.axis_index("s")
    wid = sid * NC + cid

    def fill_ones(i, _):
        ones_v[i] = jnp.full((16,), 1.0, jnp.float32)
        return _

    lax.fori_loop(0, CH, fill_ones, None)

    def fill_z(i, _):
        z_v[i] = jnp.zeros((16,), jnp.float32)
        return _

    lax.fori_loop(0, RPT, fill_z, None)
    pltpu.sync_copy(z_v, deg_sh.at[pl.ds(sid * RPT, RPT)])
    pltpu.sync_copy(dst_hbm.at[wid], dst_all)
    plsc.subcore_barrier()

    def chunk(c, _):
        pltpu.sync_copy(ones_v, deg_sh.at[dst_all.at[c]], add=True)
        return _

    lax.fori_loop(0, NCHUNK, chunk, None)
    plsc.subcore_barrier()
    pltpu.sync_copy(deg_sh.at[pl.ds(sid * RPT, RPT)],
                    out_hbm.at[cid, pl.ds(sid * RPT, RPT)])


@functools.partial(
    pl.kernel,
    mesh=_mesh,
    compiler_params=_sc_params,
    out_type=jax.ShapeDtypeStruct((NC, NPAD, D), jnp.float32),
    scratch_types=[
        pltpu.VMEM((NCHUNK, CH), jnp.int32),
        pltpu.VMEM((NCHUNK, CH), jnp.int32),
        pltpu.VMEM((NBUF, CH, D), jnp.float32),
        pltpu.VMEM_SHARED((NPAD, D), jnp.float32),
    ] + [pltpu.SemaphoreType.DMA] * NBUF,
)
def _sc_agg(p_hbm, src_hbm, dst_hbm, out_hbm,
            src_all, dst_all, rows, agg_sh, *gsems):
    cid = lax.axis_index("c")
    sid = lax.axis_index("s")
    wid = sid * NC + cid

    def fill_z(i, _):
        for j in range(D // 16):
            rows[0, i, pl.ds(16 * j, 16)] = jnp.zeros((16,), jnp.float32)
        return _

    lax.fori_loop(0, CH, fill_z, None)
    for r in range(RPT // CH):
        pltpu.sync_copy(rows.at[0], agg_sh.at[pl.ds(sid * RPT + r * CH, CH)])
    pltpu.sync_copy(src_hbm.at[wid], src_all)
    pltpu.sync_copy(dst_hbm.at[wid], dst_all)
    plsc.subcore_barrier()

    for b in range(NBUF):
        pltpu.async_copy(p_hbm.at[src_all.at[b]], rows.at[b], gsems[b])

    def group(g, _):
        for b in range(NBUF):
            c = g * NBUF + b
            pltpu.make_async_copy(
                p_hbm.at[src_all.at[c]], rows.at[b], gsems[b]).wait()
            pltpu.sync_copy(rows.at[b], agg_sh.at[dst_all.at[c]], add=True)
            nxt = c + NBUF

            @pl.when(nxt < NCHUNK)
            def _start():
                pltpu.async_copy(p_hbm.at[src_all.at[nxt]], rows.at[b],
                                 gsems[b])
        return _

    lax.fori_loop(0, NCHUNK // NBUF, group, None)
    plsc.subcore_barrier()
    pltpu.sync_copy(agg_sh.at[pl.ds(sid * RPT, RPT)],
                    out_hbm.at[cid, pl.ds(sid * RPT, RPT)])


_R = 1024


def _prep_body(x_ref, w_ref, degp_ref, p_ref, dis_ref):
    h = jnp.dot(x_ref[...], w_ref[...], preferred_element_type=jnp.float32)
    dis = lax.rsqrt(degp_ref[0] + degp_ref[1] + 1.0)
    dis_ref[...] = dis
    p_ref[...] = h * dis[:, :1]


def _mid_body(parts_ref, p1_ref, dis_ref, b1_ref, w2_ref, p2_ref):
    dis = dis_ref[...][:, :1]
    t = dis * (parts_ref[0] + parts_ref[1] + p1_ref[...]) + b1_ref[...]
    h2 = jnp.maximum(t, 0.0)
    p2_ref[...] = jnp.dot(h2, w2_ref[...],
                          preferred_element_type=jnp.float32) * dis


def _out_body(parts_ref, p2_ref, dis_ref, b2_ref, out_ref):
    dis = dis_ref[...][:, :1]
    out_ref[...] = dis * (parts_ref[0] + parts_ref[1] + p2_ref[...]) \
        + b2_ref[...]


def _tc_prep(x, W1, degp):
    grid = (NPAD // _R,)
    return pl.pallas_call(
        _prep_body,
        grid=grid,
        in_specs=[
            pl.BlockSpec((_R, D), lambda i: (i, 0)),
            pl.BlockSpec((D, D), lambda i: (0, 0)),
            pl.BlockSpec((NC, _R, 16), lambda i: (0, i, 0)),
        ],
        out_specs=[
            pl.BlockSpec((_R, D), lambda i: (i, 0)),
            pl.BlockSpec((_R, 16), lambda i: (i, 0)),
        ],
        out_shape=[
            jax.ShapeDtypeStruct((NPAD, D), jnp.float32),
            jax.ShapeDtypeStruct((NPAD, 16), jnp.float32),
        ],
    )(x, W1, degp)


def _tc_mid(parts, p1, dis16, b1, W2):
    grid = (NPAD // _R,)
    return pl.pallas_call(
        _mid_body,
        grid=grid,
        in_specs=[
            pl.BlockSpec((NC, _R, D), lambda i: (0, i, 0)),
            pl.BlockSpec((_R, D), lambda i: (i, 0)),
            pl.BlockSpec((_R, 16), lambda i: (i, 0)),
            pl.BlockSpec((1, D), lambda i: (0, 0)),
            pl.BlockSpec((D, D), lambda i: (0, 0)),
        ],
        out_specs=pl.BlockSpec((_R, D), lambda i: (i, 0)),
        out_shape=jax.ShapeDtypeStruct((NPAD, D), jnp.float32),
    )(parts, p1, dis16, b1, W2)


def _tc_out(parts, p2, dis16, b2):
    grid = (NPAD // _R,)
    return pl.pallas_call(
        _out_body,
        grid=grid,
        in_specs=[
            pl.BlockSpec((NC, _R, D), lambda i: (0, i, 0)),
            pl.BlockSpec((_R, D), lambda i: (i, 0)),
            pl.BlockSpec((_R, 16), lambda i: (i, 0)),
            pl.BlockSpec((1, D), lambda i: (0, 0)),
        ],
        out_specs=pl.BlockSpec((_R, D), lambda i: (i, 0)),
        out_shape=jax.ShapeDtypeStruct((NPAD, D), jnp.float32),
    )(parts, p2, dis16, b2)


def kernel(x, edge_index, W1, b1, W2, b2):
    ei = edge_index.astype(jnp.int32)
    pad = N_NODES + (jnp.arange(EPAD - N_EDGES, dtype=jnp.int32)
                     % (NPAD - N_NODES))
    src3 = jnp.concatenate([ei[0], pad]).reshape(NW, NCHUNK, CH)
    dst3 = jnp.concatenate([ei[1], pad]).reshape(NW, NCHUNK, CH)
    xp = jnp.concatenate(
        [x, jnp.zeros((NPAD - N_NODES, D), x.dtype)])

    degp = _sc_deg(dst3)
    p1, dis16 = _tc_prep(xp, W1, degp)
    parts1 = _sc_agg(p1, src3, dst3)
    p2 = _tc_mid(parts1, p1, dis16, b1.reshape(1, D), W2)
    parts2 = _sc_agg(p2, src3, dst3)
    return _tc_out(parts2, p2, dis16, b2.reshape(1, D))[:N_NODES]


# speedup vs baseline: 29.8147x; 1.0146x over previous
import functools

import jax
import jax.numpy as jnp
from jax import lax
from jax.experimental import pallas as pl
from jax.experimental.pallas import tpu as pltpu
from jax.experimental.pallas import tpu_sc as plsc

N_NODES = 10000
D = 128
N_EDGES = 320000

NC = 2
NS = 16
NW = NC * NS
NPAD = 10240
CH = 80
NCHUNK = 128
EPW = NCHUNK * CH
EPAD = NW * EPW
RPT = NPAD // NS
NBUF = 2

_mesh = plsc.VectorSubcoreMesh(core_axis_name="c", subcore_axis_name="s")
_sc_params = pltpu.CompilerParams(use_tc_tiling_on_sc=False)


@functools.partial(
    pl.kernel,
    mesh=_mesh,
    compiler_params=_sc_params,
    out_type=jax.ShapeDtypeStruct((NC, NPAD, 16), jnp.float32),
    scratch_types=[
        pltpu.VMEM((NCHUNK, CH), jnp.int32),
        pltpu.VMEM((CH, 16), jnp.float32),
        pltpu.VMEM((RPT, 16), jnp.float32),
        pltpu.VMEM_SHARED((NPAD, 16), jnp.float32),
    ],
)
def _sc_deg(dst_hbm, out_hbm, dst_all, ones_v, z_v, deg_sh):
    cid = lax.axis_index("c")
    sid = lax.axis_index("s")
    wid = sid * NC + cid

    def fill_ones(i, _):
        ones_v[i] = jnp.full((16,), 1.0, jnp.float32)
        return _

    lax.fori_loop(0, CH, fill_ones, None)

    def fill_z(i, _):
        z_v[i] = jnp.zeros((16,), jnp.float32)
        return _

    lax.fori_loop(0, RPT, fill_z, None)
    pltpu.sync_copy(z_v, deg_sh.at[pl.ds(sid * RPT, RPT)])
    pltpu.sync_copy(dst_hbm.at[wid], dst_all)
    plsc.subcore_barrier()

    def chunk(c, _):
        pltpu.sync_copy(ones_v, deg_sh.at[dst_all.at[c]], add=True)
        return _

    lax.fori_loop(0, NCHUNK, chunk, None)
    plsc.subcore_barrier()
    pltpu.sync_copy(deg_sh.at[pl.ds(sid * RPT, RPT)],
                    out_hbm.at[cid, pl.ds(sid * RPT, RPT)])


@functools.partial(
    pl.kernel,
    mesh=_mesh,
    compiler_params=_sc_params,
    out_type=jax.ShapeDtypeStruct((NC, NPAD, D), jnp.float32),
    scratch_types=[
        pltpu.VMEM((NCHUNK, CH), jnp.int32),
        pltpu.VMEM((NCHUNK, CH), jnp.int32),
        pltpu.VMEM((NBUF, CH, D), jnp.float32),
        pltpu.VMEM_SHARED((NPAD, D), jnp.float32),
    ] + [pltpu.SemaphoreType.DMA] * NBUF,
)
def _sc_agg(p_hbm, src_hbm, dst_hbm, out_hbm,
            src_all, dst_all, rows, agg_sh, *gsems):
    cid = lax.axis_index("c")
    sid = lax.axis_index("s")
    wid = sid * NC + cid

    def fill_z(i, _):
        for j in range(D // 16):
            rows[0, i, pl.ds(16 * j, 16)] = jnp.zeros((16,), jnp.float32)
        return _

    lax.fori_loop(0, CH, fill_z, None)
    for r in range(RPT // CH):
        pltpu.sync_copy(rows.at[0], agg_sh.at[pl.ds(sid * RPT + r * CH, CH)])
    pltpu.sync_copy(src_hbm.at[wid], src_all)
    pltpu.sync_copy(dst_hbm.at[wid], dst_all)
    plsc.subcore_barrier()

    for b in range(NBUF):
        pltpu.async_copy(p_hbm.at[src_all.at[b]], rows.at[b], gsems[b])

    def group(g, _):
        for b in range(NBUF):
            c = g * NBUF + b
            pltpu.make_async_copy(
                p_hbm.at[src_all.at[c]], rows.at[b], gsems[b]).wait()
            pltpu.sync_copy(rows.at[b], agg_sh.at[dst_all.at[c]], add=True)
            nxt = c + NBUF

            @pl.when(nxt < NCHUNK)
            def _start():
                pltpu.async_copy(p_hbm.at[src_all.at[nxt]], rows.at[b],
                                 gsems[b])
        return _

    lax.fori_loop(0, NCHUNK // NBUF, group, None)
    plsc.subcore_barrier()
    pltpu.sync_copy(agg_sh.at[pl.ds(sid * RPT, RPT)],
                    out_hbm.at[cid, pl.ds(sid * RPT, RPT)])


_R = 1024


def _mm_body(x_ref, w_ref, h_ref):
    h_ref[...] = jnp.dot(x_ref[...], w_ref[...],
                         preferred_element_type=jnp.float32)


def _prep_body(h_ref, degp_ref, p_ref, dis_ref):
    dis = lax.rsqrt(degp_ref[0] + degp_ref[1] + 1.0)
    dis_ref[...] = dis
    p_ref[...] = h_ref[...] * dis[:, :1]


def _mid_body(parts_ref, p1_ref, dis_ref, b1_ref, w2_ref, p2_ref):
    dis = dis_ref[...][:, :1]
    t = dis * (parts_ref[0] + parts_ref[1] + p1_ref[...]) + b1_ref[...]
    h2 = jnp.maximum(t, 0.0)
    p2_ref[...] = jnp.dot(h2, w2_ref[...],
                          preferred_element_type=jnp.float32) * dis


def _out_body(parts_ref, p2_ref, dis_ref, b2_ref, out_ref):
    dis = dis_ref[...][:, :1]
    out_ref[...] = dis * (parts_ref[0] + parts_ref[1] + p2_ref[...]) \
        + b2_ref[...]


def _tc_mm(x, W1):
    grid = (NPAD // _R,)
    return pl.pallas_call(
        _mm_body,
        grid=grid,
        in_specs=[
            pl.BlockSpec((_R, D), lambda i: (i, 0)),
            pl.BlockSpec((D, D), lambda i: (0, 0)),
        ],
        out_specs=pl.BlockSpec((_R, D), lambda i: (i, 0)),
        out_shape=jax.ShapeDtypeStruct((NPAD, D), jnp.float32),
    )(x, W1)


def _tc_prep(h, degp):
    grid = (NPAD // _R,)
    return pl.pallas_call(
        _prep_body,
        grid=grid,
        in_specs=[
            pl.BlockSpec((_R, D), lambda i: (i, 0)),
            pl.BlockSpec((NC, _R, 16), lambda i: (0, i, 0)),
        ],
        out_specs=[
            pl.BlockSpec((_R, D), lambda i: (i, 0)),
            pl.BlockSpec((_R, 16), lambda i: (i, 0)),
        ],
        out_shape=[
            jax.ShapeDtypeStruct((NPAD, D), jnp.float32),
            jax.ShapeDtypeStruct((NPAD, 16), jnp.float32),
        ],
    )(h, degp)


def _tc_mid(parts, p1, dis16, b1, W2):
    grid = (NPAD // _R,)
    return pl.pallas_call(
        _mid_body,
        grid=grid,
        in_specs=[
            pl.BlockSpec((NC, _R, D), lambda i: (0, i, 0)),
            pl.BlockSpec((_R, D), lambda i: (i, 0)),
            pl.BlockSpec((_R, 16), lambda i: (i, 0)),
            pl.BlockSpec((1, D), lambda i: (0, 0)),
            pl.BlockSpec((D, D), lambda i: (0, 0)),
        ],
        out_specs=pl.BlockSpec((_R, D), lambda i: (i, 0)),
        out_shape=jax.ShapeDtypeStruct((NPAD, D), jnp.float32),
    )(parts, p1, dis16, b1, W2)


def _tc_out(parts, p2, dis16, b2):
    ro = 1000
    grid = (N_NODES // ro,)
    return pl.pallas_call(
        _out_body,
        grid=grid,
        in_specs=[
            pl.BlockSpec((NC, ro, D), lambda i: (0, i, 0)),
            pl.BlockSpec((ro, D), lambda i: (i, 0)),
            pl.BlockSpec((ro, 16), lambda i: (i, 0)),
            pl.BlockSpec((1, D), lambda i: (0, 0)),
        ],
        out_specs=pl.BlockSpec((ro, D), lambda i: (i, 0)),
        out_shape=jax.ShapeDtypeStruct((N_NODES, D), jnp.float32),
    )(parts, p2, dis16, b2)


def kernel(x, edge_index, W1, b1, W2, b2):
    ei = edge_index.astype(jnp.int32)
    pad = N_NODES + (jnp.arange(EPAD - N_EDGES, dtype=jnp.int32)
                     % (NPAD - N_NODES))
    src3 = jnp.concatenate([ei[0], pad]).reshape(NW, NCHUNK, CH)
    dst3 = jnp.concatenate([ei[1], pad]).reshape(NW, NCHUNK, CH)
    xp = jnp.concatenate(
        [x, jnp.zeros((NPAD - N_NODES, D), x.dtype)])

    degp = _sc_deg(dst3)
    h1 = _tc_mm(xp, W1)
    p1, dis16 = _tc_prep(h1, degp)
    parts1 = _sc_agg(p1, src3, dst3)
    p2 = _tc_mid(parts1, p1, dis16, b1.reshape(1, D), W2)
    parts2 = _sc_agg(p2, src3, dst3)
    return _tc_out(parts2, p2, dis16, b2.reshape(1, D))


# speedup vs baseline: 36.9468x; 1.2392x over previous
import functools

import jax
import jax.numpy as jnp
from jax import lax
from jax.experimental import pallas as pl
from jax.experimental.pallas import tpu as pltpu
from jax.experimental.pallas import tpu_sc as plsc

N_NODES = 10000
D = 128
N_EDGES = 320000

NC = 2
NS = 16
NW = NC * NS
NPAD = 10240
CH = 80
NCHUNK = 128
EPW = NCHUNK * CH
EPAD = NW * EPW
RPT = NPAD // NS
NBUF = 4

_mesh = plsc.VectorSubcoreMesh(core_axis_name="c", subcore_axis_name="s")
_sc_params = pltpu.CompilerParams(use_tc_tiling_on_sc=False)


@functools.partial(
    pl.kernel,
    mesh=_mesh,
    compiler_params=_sc_params,
    out_type=jax.ShapeDtypeStruct((NC, NPAD, 16), jnp.float32),
    scratch_types=[
        pltpu.VMEM((NCHUNK, CH), jnp.int32),
        pltpu.VMEM((CH, 16), jnp.float32),
        pltpu.VMEM((RPT, 16), jnp.float32),
        pltpu.VMEM_SHARED((NPAD, 16), jnp.float32),
    ],
)
def _sc_deg(dst_hbm, out_hbm, dst_all, ones_v, z_v, deg_sh):
    cid = lax.axis_index("c")
    sid = lax.axis_index("s")
    wid = sid * NC + cid

    def fill_ones(i, _):
        ones_v[i] = jnp.full((16,), 1.0, jnp.float32)
        return _

    lax.fori_loop(0, CH, fill_ones, None)

    def fill_z(i, _):
        z_v[i] = jnp.zeros((16,), jnp.float32)
        return _

    lax.fori_loop(0, RPT, fill_z, None)
    pltpu.sync_copy(z_v, deg_sh.at[pl.ds(sid * RPT, RPT)])
    pltpu.sync_copy(dst_hbm.at[wid], dst_all)
    plsc.subcore_barrier()

    def chunk(c, _):
        pltpu.sync_copy(ones_v, deg_sh.at[dst_all.at[c]], add=True)
        return _

    lax.fori_loop(0, NCHUNK, chunk, None)
    plsc.subcore_barrier()
    pltpu.sync_copy(deg_sh.at[pl.ds(sid * RPT, RPT)],
                    out_hbm.at[cid, pl.ds(sid * RPT, RPT)])


@functools.partial(
    pl.kernel,
    mesh=_mesh,
    compiler_params=_sc_params,
    out_type=jax.ShapeDtypeStruct((NC, NPAD, D), jnp.bfloat16),
    scratch_types=[
        pltpu.VMEM((NCHUNK, CH), jnp.int32),
        pltpu.VMEM((NCHUNK, CH), jnp.int32),
        pltpu.VMEM((NBUF, CH, D), jnp.bfloat16),
        pltpu.VMEM_SHARED((NPAD, D), jnp.bfloat16),
    ] + [pltpu.SemaphoreType.DMA] * NBUF,
)
def _sc_agg(p_hbm, src_hbm, dst_hbm, out_hbm,
            src_all, dst_all, rows, agg_sh, *gsems):
    cid = lax.axis_index("c")
    sid = lax.axis_index("s")
    wid = sid * NC + cid

    def fill_z(i, _):
        for j in range(D // 32):
            rows[0, i, pl.ds(32 * j, 32)] = jnp.zeros((32,), jnp.bfloat16)
        return _

    lax.fori_loop(0, CH, fill_z, None)
    for r in range(RPT // CH):
        pltpu.sync_copy(rows.at[0], agg_sh.at[pl.ds(sid * RPT + r * CH, CH)])
    pltpu.sync_copy(src_hbm.at[wid], src_all)
    pltpu.sync_copy(dst_hbm.at[wid], dst_all)
    plsc.subcore_barrier()

    for b in range(NBUF):
        pltpu.async_copy(p_hbm.at[src_all.at[b]], rows.at[b], gsems[b])

    def group(g, _):
        for b in range(NBUF):
            c = g * NBUF + b
            pltpu.make_async_copy(
                p_hbm.at[src_all.at[c]], rows.at[b], gsems[b]).wait()
            pltpu.sync_copy(rows.at[b], agg_sh.at[dst_all.at[c]], add=True)
            nxt = c + NBUF

            @pl.when(nxt < NCHUNK)
            def _start():
                pltpu.async_copy(p_hbm.at[src_all.at[nxt]], rows.at[b],
                                 gsems[b])
        return _

    lax.fori_loop(0, NCHUNK // NBUF, group, None)
    plsc.subcore_barrier()
    pltpu.sync_copy(agg_sh.at[pl.ds(sid * RPT, RPT)],
                    out_hbm.at[cid, pl.ds(sid * RPT, RPT)])


_R = 1024


def _mm_body(x_ref, w_ref, h_ref):
    h_ref[...] = jnp.dot(x_ref[...], w_ref[...],
                         preferred_element_type=jnp.float32)


def _prep_body(h_ref, degp_ref, p_ref, pb_ref, dis_ref):
    dis = lax.rsqrt(degp_ref[0] + degp_ref[1] + 1.0)
    dis_ref[...] = dis
    p = h_ref[...] * dis[:, :1]
    p_ref[...] = p
    pb_ref[...] = p.astype(jnp.bfloat16)


def _mid_body(parts_ref, p1_ref, dis_ref, b1_ref, w2_ref, p2_ref, p2b_ref):
    dis = dis_ref[...][:, :1]
    agg = (parts_ref[0] + parts_ref[1]).astype(jnp.float32)
    t = dis * (agg + p1_ref[...]) + b1_ref[...]
    h2 = jnp.maximum(t, 0.0)
    p2 = jnp.dot(h2, w2_ref[...], preferred_element_type=jnp.float32) * dis
    p2_ref[...] = p2
    p2b_ref[...] = p2.astype(jnp.bfloat16)


def _out_body(parts_ref, p2_ref, dis_ref, b2_ref, out_ref):
    dis = dis_ref[...][:, :1]
    agg = (parts_ref[0] + parts_ref[1]).astype(jnp.float32)
    out_ref[...] = dis * (agg + p2_ref[...]) + b2_ref[...]


def _tc_mm(x, W1):
    grid = (NPAD // _R,)
    return pl.pallas_call(
        _mm_body,
        grid=grid,
        in_specs=[
            pl.BlockSpec((_R, D), lambda i: (i, 0)),
            pl.BlockSpec((D, D), lambda i: (0, 0)),
        ],
        out_specs=pl.BlockSpec((_R, D), lambda i: (i, 0)),
        out_shape=jax.ShapeDtypeStruct((NPAD, D), jnp.float32),
    )(x, W1)


def _tc_prep(h, degp):
    grid = (NPAD // _R,)
    return pl.pallas_call(
        _prep_body,
        grid=grid,
        in_specs=[
            pl.BlockSpec((_R, D), lambda i: (i, 0)),
            pl.BlockSpec((NC, _R, 16), lambda i: (0, i, 0)),
        ],
        out_specs=[
            pl.BlockSpec((_R, D), lambda i: (i, 0)),
            pl.BlockSpec((_R, D), lambda i: (i, 0)),
            pl.BlockSpec((_R, 16), lambda i: (i, 0)),
        ],
        out_shape=[
            jax.ShapeDtypeStruct((NPAD, D), jnp.float32),
            jax.ShapeDtypeStruct((NPAD, D), jnp.bfloat16),
            jax.ShapeDtypeStruct((NPAD, 16), jnp.float32),
        ],
    )(h, degp)


def _tc_mid(parts, p1, dis16, b1, W2):
    grid = (NPAD // _R,)
    return pl.pallas_call(
        _mid_body,
        grid=grid,
        in_specs=[
            pl.BlockSpec((NC, _R, D), lambda i: (0, i, 0)),
            pl.BlockSpec((_R, D), lambda i: (i, 0)),
            pl.BlockSpec((_R, 16), lambda i: (i, 0)),
            pl.BlockSpec((1, D), lambda i: (0, 0)),
            pl.BlockSpec((D, D), lambda i: (0, 0)),
        ],
        out_specs=[
            pl.BlockSpec((_R, D), lambda i: (i, 0)),
            pl.BlockSpec((_R, D), lambda i: (i, 0)),
        ],
        out_shape=[
            jax.ShapeDtypeStruct((NPAD, D), jnp.float32),
            jax.ShapeDtypeStruct((NPAD, D), jnp.bfloat16),
        ],
    )(parts, p1, dis16, b1, W2)


def _tc_out(parts, p2, dis16, b2):
    ro = 1000
    grid = (N_NODES // ro,)
    return pl.pallas_call(
        _out_body,
        grid=grid,
        in_specs=[
            pl.BlockSpec((NC, ro, D), lambda i: (0, i, 0)),
            pl.BlockSpec((ro, D), lambda i: (i, 0)),
            pl.BlockSpec((ro, 16), lambda i: (i, 0)),
            pl.BlockSpec((1, D), lambda i: (0, 0)),
        ],
        out_specs=pl.BlockSpec((ro, D), lambda i: (i, 0)),
        out_shape=jax.ShapeDtypeStruct((N_NODES, D), jnp.float32),
    )(parts, p2, dis16, b2)


def kernel(x, edge_index, W1, b1, W2, b2):
    ei = edge_index.astype(jnp.int32)
    pad = N_NODES + (jnp.arange(EPAD - N_EDGES, dtype=jnp.int32)
                     % (NPAD - N_NODES))
    src3 = jnp.concatenate([ei[0], pad]).reshape(NW, NCHUNK, CH)
    dst3 = jnp.concatenate([ei[1], pad]).reshape(NW, NCHUNK, CH)
    xp = jnp.concatenate(
        [x, jnp.zeros((NPAD - N_NODES, D), x.dtype)])

    degp = _sc_deg(dst3)
    h1 = _tc_mm(xp, W1)
    p1, p1b, dis16 = _tc_prep(h1, degp)
    parts1 = _sc_agg(p1b, src3, dst3)
    p2, p2b = _tc_mid(parts1, p1, dis16, b1.reshape(1, D), W2)
    parts2 = _sc_agg(p2b, src3, dst3)
    return _tc_out(parts2, p2, dis16, b2.reshape(1, D))
